# baseline (speedup 1.0000x reference)
import jax
import jax.numpy as jnp
from jax import lax
from jax.experimental import pallas as pl
from jax.experimental.layout import Layout, with_layout_constraint
from jax.experimental.pallas import tpu as pltpu
from jax.experimental.pallas import tpu_sc as plsc

NC, NS = 2, 16
NW = NC * NS
NBUF = 8
K = 4


def _gather_body(table_hbm, x_hbm, out_hbm, idx_v, bufs, gsem, osem):
    rows_w = x_hbm.shape[0] // NW
    ngroup = rows_w // NBUF
    wid = lax.axis_index("s") * NC + lax.axis_index("c")
    base = wid * rows_w

    pltpu.sync_copy(x_hbm.at[pl.ds(base, rows_w)], idx_v)

    def start_gather(j, b):
        return pltpu.async_copy(table_hbm.at[idx_v.at[j]], bufs.at[b], gsem.at[b])

    def start_out(j, b):
        pltpu.async_copy(bufs.at[b], out_hbm.at[base + j], osem.at[b])

    def wait_out(b):
        pltpu.make_async_copy(bufs.at[b], out_hbm.at[base], osem.at[b]).wait()

    hs = [start_gather(b, b) for b in range(NBUF)]
    for b in range(NBUF):
        hs[b].wait()
        start_out(b, b)

    def group(g, carry):
        hg = []
        for b in range(NBUF):
            wait_out(b)
            hg.append(start_gather(g * NBUF + b, b))
        for b in range(NBUF):
            hg[b].wait()
            start_out(g * NBUF + b, b)
        return carry

    lax.fori_loop(1, ngroup, group, 0)

    for b in range(NBUF):
        wait_out(b)


def _gather_slice(table, xs):
    Bs, S = xs.shape
    V, D = table.shape
    rows_w = Bs // NW
    mesh = plsc.VectorSubcoreMesh(core_axis_name="c", subcore_axis_name="s")
    return pl.kernel(
        _gather_body,
        out_type=jax.ShapeDtypeStruct((Bs, S, D), table.dtype),
        mesh=mesh,
        scratch_types=[
            pltpu.VMEM((rows_w, S), jnp.int32),
            pltpu.VMEM((NBUF, S, D), jnp.float32),
            pltpu.SemaphoreType.DMA((NBUF,)),
            pltpu.SemaphoreType.DMA((NBUF,)),
        ],
    )(table, xs)


def kernel(x, table):
    B, S = x.shape
    V, D = table.shape
    xi = x.astype(jnp.int32)
    Bs = B // K
    parts = [_gather_slice(table, lax.slice(xi, (k * Bs, 0), ((k + 1) * Bs, S)))
             for k in range(K)]
    tiled = Layout((0, 1, 2), ((8, 128),))
    acc = with_layout_constraint(jnp.zeros((B, S, D), table.dtype), tiled)
    acc = lax.optimization_barrier(acc)
    for k in range(K):
        acc = lax.dynamic_update_slice(acc, parts[k], (k * Bs, 0, 0))
        acc = with_layout_constraint(lax.optimization_barrier(acc), tiled)
    return acc


# speedup vs baseline: 1.2666x; 1.2666x over previous
import jax
import jax.numpy as jnp
from jax import lax
from jax.experimental import pallas as pl
from jax.experimental.pallas import tpu as pltpu
from jax.experimental.pallas import tpu_sc as plsc

NC, NS = 2, 16
NW = NC * NS
NBUF = 8
GPB = 2


def _gather_body(table_hbm, x_hbm, out_hbm, idx_v, bufs, gsem, osem):
    rows_w = x_hbm.shape[0] // NW
    nchunk = rows_w // GPB
    ngroup = nchunk // NBUF
    wid = lax.axis_index("s") * NC + lax.axis_index("c")
    base = wid * rows_w

    pltpu.sync_copy(x_hbm.at[pl.ds(base, rows_w)], idx_v)

    def start_gathers(j, b):
        return [pltpu.async_copy(table_hbm.at[idx_v.at[j * GPB + i]],
                                 bufs.at[b, i], gsem.at[b])
                for i in range(GPB)]

    def start_out(j, b):
        pltpu.async_copy(bufs.at[b], out_hbm.at[pl.ds(base + j * GPB, GPB)],
                         osem.at[b])

    def wait_out(b):
        pltpu.make_async_copy(bufs.at[b], out_hbm.at[pl.ds(base, GPB)],
                              osem.at[b]).wait()

    hs = [start_gathers(b, b) for b in range(NBUF)]
    for b in range(NBUF):
        for h in hs[b]:
            h.wait()
        start_out(b, b)

    def group(g, carry):
        hg = []
        for b in range(NBUF):
            wait_out(b)
            hg.append(start_gathers(g * NBUF + b, b))
        for b in range(NBUF):
            for h in hg[b]:
                h.wait()
            start_out(g * NBUF + b, b)
        return carry

    lax.fori_loop(1, ngroup, group, 0)

    for b in range(NBUF):
        wait_out(b)


def kernel(x, table):
    B, S = x.shape
    V, D = table.shape
    rows_w = B // NW
    mesh = plsc.VectorSubcoreMesh(core_axis_name="c", subcore_axis_name="s")
    out = pl.kernel(
        _gather_body,
        out_type=jax.ShapeDtypeStruct((B, S, D), table.dtype),
        mesh=mesh,
        scratch_types=[
            pltpu.VMEM((rows_w, S), jnp.int32),
            pltpu.VMEM((NBUF, GPB, S, D), jnp.float32),
            pltpu.SemaphoreType.DMA((NBUF,)),
            pltpu.SemaphoreType.DMA((NBUF,)),
        ],
    )(table, x.astype(jnp.int32))
    return out


# speedup vs baseline: 1.2723x; 1.0045x over previous
import jax
import jax.numpy as jnp
from jax import lax
from jax.experimental import pallas as pl
from jax.experimental.pallas import tpu as pltpu
from jax.experimental.pallas import tpu_sc as plsc

NC, NS = 2, 16
NW = NC * NS
NBUF = 8


def _gather_body(table_hbm, x_hbm, out_hbm, idx_v, bufs, gsem, osem):
    rows_w = x_hbm.shape[0] // NW
    ngroup = rows_w // NBUF
    wid = lax.axis_index("s") * NC + lax.axis_index("c")
    base = wid * rows_w

    pltpu.sync_copy(x_hbm.at[pl.ds(base, rows_w)], idx_v)

    def start_gather(j, b):
        return pltpu.async_copy(table_hbm.at[idx_v.at[j]], bufs.at[b], gsem.at[b])

    def start_out(j, b):
        pltpu.async_copy(bufs.at[b], out_hbm.at[base + j], osem.at[b])

    def wait_out(b):
        pltpu.make_async_copy(bufs.at[b], out_hbm.at[base], osem.at[b]).wait()

    hs = [start_gather(b, b) for b in range(NBUF)]
    for b in range(NBUF):
        hs[b].wait()
        start_out(b, b)

    def group(g, carry):
        hg = []
        for b in range(NBUF):
            wait_out(b)
            hg.append(start_gather(g * NBUF + b, b))
        for b in range(NBUF):
            hg[b].wait()
            start_out(g * NBUF + b, b)
        return carry

    lax.fori_loop(1, ngroup, group, 0)

    for b in range(NBUF):
        wait_out(b)


def kernel(x, table):
    B, S = x.shape
    V, D = table.shape
    rows_w = B // NW
    mesh = plsc.VectorSubcoreMesh(core_axis_name="c", subcore_axis_name="s")
    out = pl.kernel(
        _gather_body,
        out_type=jax.ShapeDtypeStruct((B, S, D), table.dtype),
        mesh=mesh,
        scratch_types=[
            pltpu.VMEM((rows_w, S), jnp.int32),
            pltpu.VMEM((NBUF, S, D), jnp.float32),
            pltpu.SemaphoreType.DMA((NBUF,)),
            pltpu.SemaphoreType.DMA((NBUF,)),
        ],
    )(table, x.astype(jnp.int32))
    return out
